# baseline (speedup 1.0000x reference)
import functools

import jax
import jax.numpy as jnp
from jax import lax
from jax.experimental import pallas as pl
from jax.experimental.pallas import tpu as pltpu
from jax.experimental.pallas import tpu_sc as plsc

NUM_CORES = 2
NUM_SUBCORES = 16
NUM_WORKERS = NUM_CORES * NUM_SUBCORES
CHUNK_ROWS = 32
LANES = 16
UNROLL = 16


def kernel(x, embedding_weight):
    batch, seq_len, hidden = x.shape
    rows = batch * seq_len
    seq_per_worker = seq_len // NUM_WORKERS
    num_chunks = seq_per_worker // CHUNK_ROWS
    ce = CHUNK_ROWS * hidden
    xf = x.reshape(rows * hidden)
    wf = embedding_weight.reshape(seq_len * hidden)

    @functools.partial(
        pl.kernel,
        mesh=plsc.VectorSubcoreMesh(core_axis_name="c", subcore_axis_name="s"),
        out_type=jax.ShapeDtypeStruct((rows * hidden,), x.dtype),
        scratch_types=[
            pltpu.VMEM((ce,), jnp.float32),
            pltpu.VMEM((ce,), jnp.float32),
            pltpu.VMEM((ce,), jnp.float32),
            pltpu.SemaphoreType.DMA,
            pltpu.SemaphoreType.DMA,
            pltpu.SemaphoreType.DMA,
            pltpu.SemaphoreType.DMA,
        ],
    )
    def sc_add(x_hbm, w_hbm, o_hbm, wbuf, xbuf0, xbuf1, xsem0, xsem1, osem0, osem1):
        wid = lax.axis_index("s") * NUM_CORES + lax.axis_index("c")
        srow = wid * seq_per_worker
        xbufs = (xbuf0, xbuf1)
        xsems = (xsem0, xsem1)
        osems = (osem0, osem1)

        def x_off(c, b):
            return (b * seq_len + srow + c * CHUNK_ROWS) * hidden

        def chunk(c, carry):
            woff = (srow + c * CHUNK_ROWS) * hidden
            for k in (0, 1):
                @pl.when(c > 0)
                def _drain(k=k):
                    pltpu.make_async_copy(
                        xbufs[k], o_hbm.at[pl.ds(x_off(c, k), ce)], osems[k]
                    ).wait()
                pltpu.async_copy(
                    x_hbm.at[pl.ds(x_off(c, k), ce)], xbufs[k], xsems[k]
                )
            pltpu.sync_copy(w_hbm.at[pl.ds(woff, ce)], wbuf)

            for b in range(batch):
                k = b & 1
                pltpu.make_async_copy(
                    x_hbm.at[pl.ds(x_off(c, b), ce)], xbufs[k], xsems[k]
                ).wait()

                @plsc.parallel_loop(0, ce // LANES, unroll=UNROLL)
                def _vadd(i, k=k):
                    s = pl.ds(i * LANES, LANES)
                    xbufs[k][s] = xbufs[k][s] + wbuf[s]

                pltpu.async_copy(
                    xbufs[k], o_hbm.at[pl.ds(x_off(c, b), ce)], osems[k]
                )
                if b + 2 < batch:
                    pltpu.make_async_copy(
                        xbufs[k], o_hbm.at[pl.ds(x_off(c, b), ce)], osems[k]
                    ).wait()
                    pltpu.async_copy(
                        x_hbm.at[pl.ds(x_off(c, b + 2), ce)], xbufs[k], xsems[k]
                    )
            return carry

        lax.fori_loop(0, num_chunks, chunk, 0)
        for k in (0, 1):
            pltpu.make_async_copy(
                xbufs[k], o_hbm.at[pl.ds(x_off(num_chunks - 1, 2 + k), ce)], osems[k]
            ).wait()

    out = sc_add(xf, wf)
    return out.reshape(batch, seq_len, hidden)


# speedup vs baseline: 4.5236x; 4.5236x over previous
import jax
import jax.numpy as jnp
from jax.experimental import pallas as pl
from jax.experimental.pallas import tpu as pltpu

SEQ_BLOCK = 2048


def _add_kernel(x_ref, w_ref, o_ref):
    o_ref[...] = x_ref[...] + w_ref[...][None, :, :]


def kernel(x, embedding_weight):
    batch, seq_len, hidden = x.shape
    num_blocks = seq_len // SEQ_BLOCK

    return pl.pallas_call(
        _add_kernel,
        grid=(num_blocks, batch),
        in_specs=[
            pl.BlockSpec((1, SEQ_BLOCK, hidden), lambda i, b: (b, i, 0)),
            pl.BlockSpec((SEQ_BLOCK, hidden), lambda i, b: (i, 0)),
        ],
        out_specs=pl.BlockSpec((1, SEQ_BLOCK, hidden), lambda i, b: (b, i, 0)),
        out_shape=jax.ShapeDtypeStruct(x.shape, x.dtype),
    )(x, embedding_weight)
